# baseline (device time: 80190 ns/iter reference)
import jax
import jax.numpy as jnp
from jax import lax
from jax.experimental import pallas as pl
from jax.experimental.pallas import tpu as pltpu

B = 16
NB = 128
BS = 16
H = 16
D = 64
PAGES_LOCAL = 128
KEYS = PAGES_LOCAL * BS
SCALE = D ** -0.5

O_OFF = 0
M_OFF = H * D
L_OFF = H * D + H
COMM_LANES = 1088


def kernel(Q, K, V, bt, lens):
    Q2 = Q.reshape(B, H * D)
    K2 = K.reshape(KEYS, H * D)
    V2 = V.reshape(KEYS, H * D)
    lens2 = lens.reshape(B, 1)

    def body(q_ref, k_ref, v_ref, bt_ref, lens_ref, out_ref,
             comm_ref, send_sem, recv_sem):
        my_x = lax.axis_index("x")
        my_y = lax.axis_index("y")
        nbr = (my_x, 1 - my_y)

        gid = (lax.broadcasted_iota(jnp.int32, (B, PAGES_LOCAL), 1)
               + my_y * PAGES_LOCAL)
        lens_v = lens_ref[:, :]
        cnt = jnp.zeros((B, PAGES_LOCAL), jnp.float32)
        for j in range(NB):
            hit = (bt_ref[:, j:j + 1] == gid) & (j < lens_v)
            cnt = cnt + hit.astype(jnp.float32)

        krow = lax.broadcasted_iota(jnp.int32, (PAGES_LOCAL, KEYS), 0)
        kcol = lax.broadcasted_iota(jnp.int32, (PAGES_LOCAL, KEYS), 1)
        expand = (kcol // BS == krow).astype(jnp.float32)
        c_key = lax.dot_general(cnt, expand, (((1,), (0,)), ((), ())),
                                preferred_element_type=jnp.float32)

        qb = q_ref[:, :].astype(jnp.bfloat16)
        kb = k_ref[:, :].astype(jnp.bfloat16)
        vb = v_ref[:, :].astype(jnp.bfloat16)
        headsel = lax.broadcasted_iota(jnp.int32, (B, H * D), 1) // D
        for h in range(H):
            q_h = jnp.where(headsel == h, qb, jnp.bfloat16(0.0))
            s = lax.dot_general(q_h, kb, (((1,), (1,)), ((), ())),
                                preferred_element_type=jnp.float32) * SCALE
            m_h = jnp.max(s, axis=1, keepdims=True)
            w = jnp.exp(s - m_h) * c_key
            l_h = jnp.sum(w, axis=1, keepdims=True)
            o_full = lax.dot_general(w.astype(jnp.bfloat16), vb,
                                     (((1,), (0,)), ((), ())),
                                     preferred_element_type=jnp.float32)
            o_h = o_full[:, h * D:(h + 1) * D]
            comm_ref[0, :, O_OFF + h * D:O_OFF + (h + 1) * D] = o_h
            comm_ref[0, :, M_OFF + h:M_OFF + h + 1] = m_h
            comm_ref[0, :, L_OFF + h:L_OFF + h + 1] = l_h

        barrier = pltpu.get_barrier_semaphore()
        pl.semaphore_signal(barrier, 1, device_id=nbr,
                            device_id_type=pl.DeviceIdType.MESH)
        pl.semaphore_wait(barrier, 1)

        rdma = pltpu.make_async_remote_copy(
            src_ref=comm_ref.at[0],
            dst_ref=comm_ref.at[1],
            send_sem=send_sem,
            recv_sem=recv_sem,
            device_id=nbr,
            device_id_type=pl.DeviceIdType.MESH,
        )
        rdma.start()
        rdma.wait()

        m_a = comm_ref[0, :, M_OFF:M_OFF + H]
        l_a = comm_ref[0, :, L_OFF:L_OFF + H]
        m_b = comm_ref[1, :, M_OFF:M_OFF + H]
        l_b = comm_ref[1, :, L_OFF:L_OFF + H]
        m_n = jnp.maximum(m_a, m_b)
        alpha = jnp.exp(m_a - m_n)
        beta = jnp.exp(m_b - m_n)
        inv_l = 1.0 / (l_a * alpha + l_b * beta)
        for h in range(H):
            o_a = comm_ref[0, :, O_OFF + h * D:O_OFF + (h + 1) * D]
            o_b = comm_ref[1, :, O_OFF + h * D:O_OFF + (h + 1) * D]
            o = (o_a * alpha[:, h:h + 1] + o_b * beta[:, h:h + 1])
            out_ref[:, h * D:(h + 1) * D] = o * inv_l[:, h:h + 1]

    out = pl.pallas_call(
        body,
        out_shape=jax.ShapeDtypeStruct((B, H * D), jnp.float32),
        in_specs=[pl.BlockSpec(memory_space=pltpu.VMEM)] * 5,
        out_specs=pl.BlockSpec(memory_space=pltpu.VMEM),
        scratch_shapes=[
            pltpu.VMEM((2, B, COMM_LANES), jnp.float32),
            pltpu.SemaphoreType.DMA,
            pltpu.SemaphoreType.DMA,
        ],
        compiler_params=pltpu.CompilerParams(collective_id=0),
    )(Q2, K2, V2, bt, lens2)
    return out.reshape(B, 1, H, D)


# device time: 14089 ns/iter; 5.6917x vs baseline; 5.6917x over previous
import jax
import jax.numpy as jnp
from jax import lax
from jax.experimental import pallas as pl
from jax.experimental.pallas import tpu as pltpu

B = 16
NB = 128
BS = 16
H = 16
HL = H // 2
D = 64
PAGES = 128
KEYS = PAGES * BS
SCALE = D ** -0.5

O_OFF = 0
M_OFF = HL * D
L_OFF = HL * D + HL
COMM_LANES = 640


def kernel(Q, K, V, bt, lens):
    Kt = jnp.transpose(K, (1, 2, 3, 0))
    Vt = jnp.transpose(V, (1, 2, 3, 0))
    slot = jnp.arange(NB, dtype=jnp.int32)[None, :]
    bt_m = jnp.where(slot < lens[:, None], bt, -1)

    def body(q_ref, kt_ref, vt_ref, bt_ref, out_ref,
             kbuf, vbuf, slab_sems, comm_ref, send_sems, recv_sems):
        my_x = lax.axis_index("x")
        my_y = lax.axis_index("y")
        my_id = my_x * 2 + my_y
        peers = [
            ((my_x, 1 - my_y), my_x * 2 + (1 - my_y)),
            ((1 - my_x, my_y), (1 - my_x) * 2 + my_y),
            ((1 - my_x, 1 - my_y), (1 - my_x) * 2 + (1 - my_y)),
        ]
        h0 = my_x * HL

        copies = []
        for t in range(BS):
            ck = pltpu.make_async_copy(
                kt_ref.at[t, pl.ds(h0, HL)],
                kbuf.at[:, :, pl.ds(t * PAGES, PAGES)],
                slab_sems.at[0, t])
            cv = pltpu.make_async_copy(
                vt_ref.at[t, pl.ds(h0, HL)],
                vbuf.at[:, :, pl.ds(t * PAGES, PAGES)],
                slab_sems.at[1, t])
            ck.start()
            cv.start()
            copies += [ck, cv]

        gid = (lax.broadcasted_iota(jnp.int32, (B, PAGES), 1) + my_y * PAGES)
        acc = [jnp.zeros((B, PAGES), jnp.float32) for _ in range(4)]
        for j in range(NB):
            acc[j % 4] = acc[j % 4] + (bt_ref[:, j:j + 1] == gid).astype(
                jnp.float32)
        cnt = (acc[0] + acc[1]) + (acc[2] + acc[3])
        c_key = jnp.concatenate([cnt] * BS, axis=1)

        barrier = pltpu.get_barrier_semaphore()
        for nbr, _ in peers:
            pl.semaphore_signal(barrier, 1, device_id=nbr,
                                device_id_type=pl.DeviceIdType.MESH)
        pl.semaphore_wait(barrier, 3)

        for c in copies:
            c.wait()

        q_half = q_ref[:, 0, pl.ds(h0, HL), :] * SCALE
        for i in range(HL):
            q_h = q_half[:, i, :]
            s = lax.dot_general(q_h, kbuf[i], (((1,), (0,)), ((), ())),
                                preferred_element_type=jnp.float32)
            m_h = jnp.max(s, axis=1, keepdims=True)
            w = jnp.exp(s - m_h) * c_key
            l_h = jnp.sum(w, axis=1, keepdims=True)
            o_h = lax.dot_general(w, vbuf[i], (((1,), (1,)), ((), ())),
                                  preferred_element_type=jnp.float32)
            comm_ref[my_id, :, O_OFF + i * D:O_OFF + (i + 1) * D] = o_h
            comm_ref[my_id, :, M_OFF + i:M_OFF + i + 1] = m_h
            comm_ref[my_id, :, L_OFF + i:L_OFF + i + 1] = l_h

        sends = []
        for idx, (nbr, _) in enumerate(peers):
            rdma = pltpu.make_async_remote_copy(
                src_ref=comm_ref.at[my_id],
                dst_ref=comm_ref.at[my_id],
                send_sem=send_sems.at[idx],
                recv_sem=recv_sems.at[my_id],
                device_id=nbr, device_id_type=pl.DeviceIdType.MESH)
            rdma.start()
            sends.append(rdma)
        for idx, (nbr, pid) in enumerate(peers):
            recv = pltpu.make_async_remote_copy(
                src_ref=comm_ref.at[my_id],
                dst_ref=comm_ref.at[pid],
                send_sem=send_sems.at[idx],
                recv_sem=recv_sems.at[pid],
                device_id=nbr, device_id_type=pl.DeviceIdType.MESH)
            recv.wait_recv()
        for rdma in sends:
            rdma.wait_send()

        for half in range(2):
            a, b = 2 * half, 2 * half + 1
            m_a = comm_ref[a, :, M_OFF:M_OFF + HL]
            l_a = comm_ref[a, :, L_OFF:L_OFF + HL]
            m_b = comm_ref[b, :, M_OFF:M_OFF + HL]
            l_b = comm_ref[b, :, L_OFF:L_OFF + HL]
            m_n = jnp.maximum(m_a, m_b)
            alpha = jnp.exp(m_a - m_n)
            beta = jnp.exp(m_b - m_n)
            inv_l = 1.0 / (l_a * alpha + l_b * beta)
            for i in range(HL):
                h = half * HL + i
                o_a = comm_ref[a, :, O_OFF + i * D:O_OFF + (i + 1) * D]
                o_b = comm_ref[b, :, O_OFF + i * D:O_OFF + (i + 1) * D]
                o = (o_a * alpha[:, i:i + 1] + o_b * beta[:, i:i + 1])
                out_ref[:, h * D:(h + 1) * D] = o * inv_l[:, i:i + 1]

    out = pl.pallas_call(
        body,
        out_shape=jax.ShapeDtypeStruct((B, H * D), jnp.float32),
        in_specs=[
            pl.BlockSpec(memory_space=pltpu.VMEM),
            pl.BlockSpec(memory_space=pl.ANY),
            pl.BlockSpec(memory_space=pl.ANY),
            pl.BlockSpec(memory_space=pltpu.VMEM),
        ],
        out_specs=pl.BlockSpec(memory_space=pltpu.VMEM),
        scratch_shapes=[
            pltpu.VMEM((HL, D, KEYS), jnp.float32),
            pltpu.VMEM((HL, D, KEYS), jnp.float32),
            pltpu.SemaphoreType.DMA((2, BS)),
            pltpu.VMEM((4, B, COMM_LANES), jnp.float32),
            pltpu.SemaphoreType.DMA((3,)),
            pltpu.SemaphoreType.DMA((4,)),
        ],
        compiler_params=pltpu.CompilerParams(collective_id=0),
    )(
        Q,
        pltpu.with_memory_space_constraint(Kt, pltpu.MemorySpace.HBM),
        pltpu.with_memory_space_constraint(Vt, pltpu.MemorySpace.HBM),
        bt_m,
    )
    return out.reshape(B, 1, H, D)


# device time: 13207 ns/iter; 6.0718x vs baseline; 1.0668x over previous
import jax
import jax.numpy as jnp
from jax import lax
from jax.experimental import pallas as pl
from jax.experimental.pallas import tpu as pltpu

B = 16
NB = 128
BS = 16
H = 16
HL = H // 2
D = 64
PAGES = 128
KEYS = PAGES * BS
SCALE = D ** -0.5

O_OFF = 0
M_OFF = HL * D
L_OFF = HL * D + HL
COMM_LANES = 640
HALF_O = HL // 2 * D
CHUNKS = ((0, HALF_O), (HALF_O, COMM_LANES - HALF_O))


def kernel(Q, K, V, bt, lens):
    Kt = jnp.transpose(K, (1, 2, 3, 0))
    Vt = jnp.transpose(V, (1, 2, 3, 0))
    slot = jnp.arange(NB, dtype=jnp.int32)[None, :]
    bt_m = jnp.where(slot < lens[:, None], bt, -1)

    def body(q_ref, kt_ref, vt_ref, bt_ref, out_ref,
             kbuf, vbuf, slab_sems, comm_ref, send_sems, recv_sems):
        my_x = lax.axis_index("x")
        my_y = lax.axis_index("y")
        my_id = my_x * 2 + my_y
        peers = [
            ((my_x, 1 - my_y), my_x * 2 + (1 - my_y)),
            ((1 - my_x, my_y), (1 - my_x) * 2 + my_y),
            ((1 - my_x, 1 - my_y), (1 - my_x) * 2 + (1 - my_y)),
        ]
        h0 = my_x * HL

        HG = HL // 2
        copies = [[], []]
        for g in range(2):
            for t in range(BS):
                ck = pltpu.make_async_copy(
                    kt_ref.at[t, pl.ds(h0 + g * HG, HG)],
                    kbuf.at[pl.ds(g * HG, HG), :, pl.ds(t * PAGES, PAGES)],
                    slab_sems.at[0, t, g])
                cv = pltpu.make_async_copy(
                    vt_ref.at[t, pl.ds(h0 + g * HG, HG)],
                    vbuf.at[pl.ds(g * HG, HG), :, pl.ds(t * PAGES, PAGES)],
                    slab_sems.at[1, t, g])
                ck.start()
                cv.start()
                copies[g] += [ck, cv]

        gid = (lax.broadcasted_iota(jnp.int32, (B, PAGES), 1) + my_y * PAGES)
        acc = [jnp.zeros((B, PAGES), jnp.float32) for _ in range(4)]
        for j in range(NB):
            acc[j % 4] = acc[j % 4] + (bt_ref[:, j:j + 1] == gid).astype(
                jnp.float32)
        cnt = (acc[0] + acc[1]) + (acc[2] + acc[3])
        c_key = jnp.concatenate([cnt] * BS, axis=1)

        barrier = pltpu.get_barrier_semaphore()
        for nbr, _ in peers:
            pl.semaphore_signal(barrier, 1, device_id=nbr,
                                device_id_type=pl.DeviceIdType.MESH)
        pl.semaphore_wait(barrier, 3)

        def start_chunk(c):
            out = []
            for idx, (nbr, _) in enumerate(peers):
                rdma = pltpu.make_async_remote_copy(
                    src_ref=comm_ref.at[my_id, :, pl.ds(*CHUNKS[c])],
                    dst_ref=comm_ref.at[my_id, :, pl.ds(*CHUNKS[c])],
                    send_sem=send_sems.at[c, idx],
                    recv_sem=recv_sems.at[c, my_id],
                    device_id=nbr, device_id_type=pl.DeviceIdType.MESH)
                rdma.start()
                out.append(rdma)
            return out

        sends = []
        q_half = q_ref[:, 0, pl.ds(h0, HL), :] * SCALE
        for i in range(HL):
            if i % HG == 0:
                for c in copies[i // HG]:
                    c.wait()
            q_h = q_half[:, i, :]
            s = lax.dot_general(q_h, kbuf[i], (((1,), (0,)), ((), ())),
                                preferred_element_type=jnp.float32)
            m_h = jnp.max(s, axis=1, keepdims=True)
            w = jnp.exp(s - m_h) * c_key
            l_h = jnp.sum(w, axis=1, keepdims=True)
            o_h = lax.dot_general(w, vbuf[i], (((1,), (1,)), ((), ())),
                                  preferred_element_type=jnp.float32)
            comm_ref[my_id, :, O_OFF + i * D:O_OFF + (i + 1) * D] = o_h
            comm_ref[my_id, :, M_OFF + i:M_OFF + i + 1] = m_h
            comm_ref[my_id, :, L_OFF + i:L_OFF + i + 1] = l_h
            if i == HL // 2 - 1:
                sends += start_chunk(0)
        sends += start_chunk(1)

        def wait_chunks(which):
            for c, (idx, (nbr, pid)) in [(c, p) for c in range(2)
                                         for p in enumerate(peers)
                                         if p[0] in which]:
                recv = pltpu.make_async_remote_copy(
                    src_ref=comm_ref.at[my_id, :, pl.ds(*CHUNKS[c])],
                    dst_ref=comm_ref.at[pid, :, pl.ds(*CHUNKS[c])],
                    send_sem=send_sems.at[c, idx],
                    recv_sem=recv_sems.at[c, pid],
                    device_id=nbr, device_id_type=pl.DeviceIdType.MESH)
                recv.wait_recv()

        def combine_pair(ids, hslices):
            a, b = ids
            m_a = comm_ref[a, :, M_OFF:M_OFF + HL]
            l_a = comm_ref[a, :, L_OFF:L_OFF + HL]
            m_b = comm_ref[b, :, M_OFF:M_OFF + HL]
            l_b = comm_ref[b, :, L_OFF:L_OFF + HL]
            m_n = jnp.maximum(m_a, m_b)
            alpha = jnp.exp(m_a - m_n)
            beta = jnp.exp(m_b - m_n)
            inv_l = 1.0 / (l_a * alpha + l_b * beta)
            parts = []
            for i in range(HL):
                o_a = comm_ref[a, :, O_OFF + i * D:O_OFF + (i + 1) * D]
                o_b = comm_ref[b, :, O_OFF + i * D:O_OFF + (i + 1) * D]
                o = (o_a * alpha[:, i:i + 1] + o_b * beta[:, i:i + 1])
                parts.append(o * inv_l[:, i:i + 1])
            off = pl.multiple_of(hslices, 128)
            out_ref[:, pl.ds(off, HL * D)] = jnp.concatenate(parts, axis=1)

        y_id = my_x * 2 + (1 - my_y)
        wait_chunks({0})
        combine_pair((my_id, y_id), my_x * (HL * D))
        wait_chunks({1, 2})
        x_id = (1 - my_x) * 2 + my_y
        d_id = (1 - my_x) * 2 + (1 - my_y)
        combine_pair((x_id, d_id), (1 - my_x) * (HL * D))
        for rdma in sends:
            rdma.wait_send()

    out = pl.pallas_call(
        body,
        out_shape=jax.ShapeDtypeStruct((B, H * D), jnp.float32),
        in_specs=[
            pl.BlockSpec(memory_space=pltpu.VMEM),
            pl.BlockSpec(memory_space=pl.ANY),
            pl.BlockSpec(memory_space=pl.ANY),
            pl.BlockSpec(memory_space=pltpu.VMEM),
        ],
        out_specs=pl.BlockSpec(memory_space=pltpu.VMEM),
        scratch_shapes=[
            pltpu.VMEM((HL, D, KEYS), jnp.float32),
            pltpu.VMEM((HL, D, KEYS), jnp.float32),
            pltpu.SemaphoreType.DMA((2, BS, 2)),
            pltpu.VMEM((4, B, COMM_LANES), jnp.float32),
            pltpu.SemaphoreType.DMA((2, 3)),
            pltpu.SemaphoreType.DMA((2, 4)),
        ],
        compiler_params=pltpu.CompilerParams(collective_id=0),
    )(
        Q,
        pltpu.with_memory_space_constraint(Kt, pltpu.MemorySpace.HBM),
        pltpu.with_memory_space_constraint(Vt, pltpu.MemorySpace.HBM),
        bt_m,
    )
    return out.reshape(B, 1, H, D)
